# baseline (speedup 1.0000x reference)
import functools

import jax
import jax.numpy as jnp
from jax import lax
from jax.experimental import pallas as pl
from jax.experimental.pallas import tpu as pltpu
from jax.experimental.pallas import tpu_sc as plsc

B_, S_, D_, F_, E_, K_ = 2, 2048, 1024, 4096, 8, 2
N_ = B_ * S_
NK = N_ * K_
CAP = int(2.0 * N_ / E_)
FB = 1024
NFB = F_ // FB
DUMP = NK


_TB = 1024


def _router_body(x_ref, wr_ref, idx_ref, w_ref, lb_ref, cnt_acc):
    i = pl.program_id(0)
    x = x_ref[...]
    wr = wr_ref[...]
    logits = lax.dot_general(x, wr, (((1,), (1,)), ((), ())),
                             preferred_element_type=jnp.float32)
    iota = lax.broadcasted_iota(jnp.int32, logits.shape, 1)
    l1 = jnp.max(logits, axis=1, keepdims=True)
    am1 = jnp.min(jnp.where(logits == l1, iota, E_), axis=1, keepdims=True)
    masked = jnp.where(iota == am1, -jnp.inf, logits)
    l2 = jnp.max(masked, axis=1, keepdims=True)
    am2 = jnp.min(jnp.where(masked == l2, iota, E_), axis=1, keepdims=True)
    z = jnp.exp(l2 - l1)
    s = 1.0 / (1.0 + z)
    idx_ref[...] = jnp.concatenate([am1, am2], axis=1)
    w_ref[...] = jnp.concatenate([s, z * s], axis=1)
    oh = (iota == am1).astype(jnp.float32) + (iota == am2).astype(jnp.float32)
    c = jnp.sum(oh, axis=0, keepdims=True)

    @pl.when(i == 0)
    def _():
        cnt_acc[...] = c

    @pl.when(i > 0)
    def _():
        cnt_acc[...] += c

    @pl.when(i == pl.num_programs(0) - 1)
    def _():
        cc = cnt_acc[...]
        ideal = jnp.float32(N_ * K_ / E_)
        lb_ref[...] = (jnp.sum((cc - ideal) ** 2) / jnp.float32(N_) ** 2
                       ).reshape(1, 1)


def _router(xf, Wr):
    return pl.pallas_call(
        _router_body,
        grid=(N_ // _TB,),
        in_specs=[pl.BlockSpec((_TB, D_), lambda i: (i, 0)),
                  pl.BlockSpec((E_, D_), lambda i: (0, 0))],
        out_specs=[pl.BlockSpec((_TB, K_), lambda i: (i, 0)),
                   pl.BlockSpec((_TB, K_), lambda i: (i, 0)),
                   pl.BlockSpec((1, 1), lambda i: (0, 0))],
        out_shape=[jax.ShapeDtypeStruct((N_, K_), jnp.int32),
                   jax.ShapeDtypeStruct((N_, K_), jnp.float32),
                   jax.ShapeDtypeStruct((1, 1), jnp.float32)],
        scratch_shapes=[pltpu.VMEM((1, E_), jnp.float32)],
    )(xf, Wr)


_CH = NK // 16
_ZSPAN = 528
_SH = 16 * _ZSPAN


_GR = 32
_GPT = NK // 32


def _dispatch_body(idx_hbm, w_hbm, xf_hbm, xg_hbm, w_out, slot_out,
                   ich, wch, dvals, rvals, svals, cvm, cnt_all, zvi, zvf,
                   gidx, gbuf, gs0, gs1, osem,
                   sh_cnt, sh_rows, sh_w):
    cid = lax.axis_index("c")
    wid = lax.axis_index("s")
    base_slot = wid * _CH

    pltpu.sync_copy(idx_hbm.at[pl.ds(base_slot, _CH)], ich)
    pltpu.sync_copy(w_hbm.at[pl.ds(base_slot, _CH)], wch)

    lanes = lax.iota(jnp.int32, 16)

    def hist_step(i, cnt):
        v = ich[pl.ds(i * 16, 16)]
        for e in range(E_):
            pc = jnp.sum((v == e).astype(jnp.int32))
            cnt = cnt + jnp.where(lanes == e, pc, 0)
        return cnt

    cnt = lax.fori_loop(0, _CH // 16, hist_step, jnp.zeros((16,), jnp.int32))
    cvm[...] = cnt
    pltpu.sync_copy(cvm, sh_cnt.at[pl.ds(wid * 16, 16)])

    def zfill(k, _):
        zvi[pl.ds(k * 16, 16)] = jnp.zeros((16,), jnp.int32)
        zvf[pl.ds(k * 16, 16)] = jnp.zeros((16,), jnp.float32)
        return 0

    lax.fori_loop(0, _ZSPAN // 16, zfill, 0)
    pltpu.sync_copy(zvi, sh_rows.at[pl.ds(wid * _ZSPAN, _ZSPAN)])
    pltpu.sync_copy(zvf, sh_w.at[pl.ds(wid * _ZSPAN, _ZSPAN)])

    plsc.subcore_barrier()

    pltpu.sync_copy(sh_cnt, cnt_all)

    base_v = jnp.zeros((16,), jnp.int32)
    for t in range(16):
        base_v = base_v + jnp.where(t < wid, cnt_all[pl.ds(t * 16, 16)], 0)
    offs = tuple(jnp.sum(jnp.where(lanes == e, base_v, 0)) for e in range(E_))

    def pc_step(i, offs):
        v = ich[pl.ds(i * 16, 16)]
        rank = jnp.zeros((16,), jnp.int32)
        new = []
        for e in range(E_):
            m = v == e
            mi = m.astype(jnp.int32)
            cs = plsc.cumsum(mi)
            rank = rank + jnp.where(m, cs - 1 + offs[e], 0)
            new.append(offs[e] + jnp.sum(mi))
        keep = rank < CAP
        dest = jnp.where(keep, v * CAP + rank, DUMP)
        dvals[i // 8, pl.ds((i % 8) * 16, 16)] = dest
        svals[pl.ds(i * 16, 16)] = dest
        slotid = base_slot + i * 16 + lanes
        rvals[pl.ds(i * 16, 16)] = lax.shift_right_logical(slotid, 1)
        return tuple(new)

    lax.fori_loop(0, _CH // 16, pc_step, offs)

    @pl.when(cid == 0)
    def _():
        pltpu.sync_copy(svals, slot_out.at[pl.ds(base_slot, _CH)])

    plsc.subcore_barrier()
    for j in range(_CH // 128):
        pltpu.sync_copy(rvals.at[pl.ds(j * 128, 128)],
                        sh_rows.at[dvals.at[j]], add=True)
        pltpu.sync_copy(wch.at[pl.ds(j * 128, 128)],
                        sh_w.at[dvals.at[j]], add=True)
    plsc.subcore_barrier()

    @pl.when(cid == 0)
    def _():
        pltpu.sync_copy(sh_w.at[pl.ds(base_slot, _CH)],
                        w_out.at[pl.ds(base_slot, _CH)])

    gwid = wid * 2 + cid
    gbase = gwid * _GPT
    nch = _GPT // _GR
    gsem = (gs0, gs1)

    def start_gather(j):
        k = j % 2
        pltpu.sync_copy(sh_rows.at[pl.ds(gbase + j * _GR, _GR)], gidx.at[k])
        pltpu.async_copy(xf_hbm.at[gidx.at[k]], gbuf.at[k], gsem[k])

    def out_desc(j):
        k = j % 2
        return pltpu.make_async_copy(
            gbuf.at[k], xg_hbm.at[pl.ds(gbase + j * _GR, _GR)], osem)

    start_gather(0)
    for j in range(nch):
        k = j % 2
        if j + 1 < nch:
            if j >= 1:
                out_desc(j - 1).wait()
            start_gather(j + 1)
        pltpu.make_async_copy(xf_hbm.at[gidx.at[k]], gbuf.at[k],
                              gsem[k]).wait()
        pltpu.async_copy(gbuf.at[k], xg_hbm.at[pl.ds(gbase + j * _GR, _GR)],
                         osem)
    out_desc(nch - 2).wait()
    out_desc(nch - 1).wait()


def _dispatch(idx_flat, w_flat, xf):
    mesh = plsc.VectorSubcoreMesh(core_axis_name="c", subcore_axis_name="s",
                                  num_cores=2, num_subcores=16)
    f = pl.kernel(
        _dispatch_body,
        out_type=[jax.ShapeDtypeStruct((NK, D_), jnp.float32),
                  jax.ShapeDtypeStruct((NK,), jnp.float32),
                  jax.ShapeDtypeStruct((NK,), jnp.int32)],
        mesh=mesh,
        compiler_params=pltpu.CompilerParams(needs_layout_passes=False),
        scratch_types=[
            pltpu.VMEM((_CH,), jnp.int32),
            pltpu.VMEM((_CH,), jnp.float32),
            pltpu.VMEM((_CH // 128, 128), jnp.int32),
            pltpu.VMEM((_CH,), jnp.int32),
            pltpu.VMEM((_CH,), jnp.int32),
            pltpu.VMEM((16,), jnp.int32),
            pltpu.VMEM((256,), jnp.int32),
            pltpu.VMEM((_ZSPAN,), jnp.int32),
            pltpu.VMEM((_ZSPAN,), jnp.float32),
            pltpu.VMEM((2, _GR), jnp.int32),
            pltpu.VMEM((2, _GR, D_), jnp.float32),
            pltpu.SemaphoreType.DMA,
            pltpu.SemaphoreType.DMA,
            pltpu.SemaphoreType.DMA,
            pltpu.VMEM_SHARED((256,), jnp.int32),
            pltpu.VMEM_SHARED((_SH,), jnp.int32),
            pltpu.VMEM_SHARED((_SH,), jnp.float32),
        ],
    )
    return f(idx_flat, w_flat, xf)


_SQRT_HALF = 0.7071067811865476


_FB2 = 2048
_NFB2 = F_ // _FB2
_NSUB = 8
_SB = _FB2 // _NSUB


def _ffn_body(xg_ref, w1_ref, b1_ref, w2_ref, b2_ref, ws_ref, og_ref, acc):
    fb = pl.program_id(1)
    x = xg_ref[...].astype(jnp.bfloat16)
    part = None
    for sub in range(_NSUB):
        w1 = w1_ref[0, pl.ds(sub * _SB, _SB), :].astype(jnp.bfloat16)
        h = lax.dot_general(x, w1, (((1,), (1,)), ((), ())),
                            preferred_element_type=jnp.float32)
        h = h + b1_ref[0, :, pl.ds(sub * _SB, _SB)]
        h = 0.5 * h * (1.0 + lax.erf(h * _SQRT_HALF))
        w2 = w2_ref[0, :, pl.ds(sub * _SB, _SB)].astype(jnp.bfloat16)
        p = lax.dot_general(h.astype(jnp.bfloat16), w2,
                            (((1,), (1,)), ((), ())),
                            preferred_element_type=jnp.float32)
        part = p if part is None else part + p

    @pl.when(fb == 0)
    def _():
        acc[...] = part

    @pl.when(fb == _NFB2 - 1)
    def _():
        og_ref[...] = (acc[...] + part + b2_ref[0]) * ws_ref[...]


def _ffn(xg, W1, b1, W2, b2, ws):
    return pl.pallas_call(
        _ffn_body,
        grid=(E_, _NFB2),
        in_specs=[
            pl.BlockSpec((CAP, D_), lambda e, fb: (e, 0)),
            pl.BlockSpec((1, _FB2, D_), lambda e, fb: (e, fb, 0)),
            pl.BlockSpec((1, 1, _FB2), lambda e, fb: (e * _NFB2 + fb, 0, 0)),
            pl.BlockSpec((1, D_, _FB2), lambda e, fb: (e, 0, fb)),
            pl.BlockSpec((1, 1, D_), lambda e, fb: (e, 0, 0)),
            pl.BlockSpec((CAP, 1), lambda e, fb: (e, 0)),
        ],
        out_specs=pl.BlockSpec((CAP, D_), lambda e, fb: (e, 0)),
        out_shape=jax.ShapeDtypeStruct((NK, D_), jnp.float32),
        scratch_shapes=[pltpu.VMEM((CAP, D_), jnp.float32)],
        compiler_params=pltpu.CompilerParams(
            dimension_semantics=("arbitrary", "arbitrary"),
            vmem_limit_bytes=63 * 1024 * 1024),
    )(xg, W1, b1.reshape(E_ * _NFB2, 1, _FB2), W2, b2.reshape(E_, 1, D_), ws)


_TPT = N_ // 32
_TCH = 16


def _combine_body(slots_hbm, og_hbm, out_hbm, sbufr, idxv, buf, obuf,
                  gs0, gs1, osem):
    cid = lax.axis_index("c")
    sid = lax.axis_index("s")
    wid = sid * 2 + cid
    tbase = wid * _TPT
    lanes = lax.iota(jnp.int32, 16)
    nch = _TPT // _TCH
    gsem = (gs0, gs1)

    def start_gather(j):
        k = j % 2
        sb = (tbase + j * _TCH) * 2
        pltpu.sync_copy(slots_hbm.at[pl.ds(sb, 2 * _TCH)], sbufr.at[k])
        sv0 = sbufr[k, pl.ds(0, 16)]
        sv1 = sbufr[k, pl.ds(16, 16)]
        idxv[k, pl.ds(0, 16)] = jnp.minimum(sv0, NK - 1)
        idxv[k, pl.ds(16, 16)] = jnp.minimum(sv1, NK - 1)
        pltpu.async_copy(og_hbm.at[idxv.at[k]], buf.at[k], gsem[k])

    def out_desc(j):
        k = j % 2
        return pltpu.make_async_copy(
            obuf.at[k], out_hbm.at[pl.ds(tbase + j * _TCH, _TCH)], osem)

    start_gather(0)
    for j in range(nch):
        k = j % 2
        if j + 1 < nch:
            start_gather(j + 1)
        pltpu.make_async_copy(og_hbm.at[idxv.at[k]], buf.at[k], gsem[k]).wait()
        if j >= 2:
            out_desc(j - 2).wait()
        sv0 = sbufr[k, pl.ds(0, 16)]
        sv1 = sbufr[k, pl.ds(16, 16)]

        def token(t, carry):
            p0 = 2 * t
            p1 = 2 * t + 1
            s0 = jnp.where(
                t < 8,
                jnp.max(jnp.where(lanes == p0, sv0, -1)),
                jnp.max(jnp.where(lanes == p0 - 16, sv1, -1)))
            s1 = jnp.where(
                t < 8,
                jnp.max(jnp.where(lanes == p1, sv0, -1)),
                jnp.max(jnp.where(lanes == p1 - 16, sv1, -1)))
            b0 = s0 < NK
            b1 = s1 < NK
            for c in range(D_ // 16):
                r0 = buf[k, p0, pl.ds(c * 16, 16)]
                r1 = buf[k, p1, pl.ds(c * 16, 16)]
                obuf[k, t, pl.ds(c * 16, 16)] = (
                    jnp.where(b0, r0, 0.0) + jnp.where(b1, r1, 0.0))
            return carry

        lax.fori_loop(0, _TCH, token, 0)
        pltpu.async_copy(obuf.at[k],
                         out_hbm.at[pl.ds(tbase + j * _TCH, _TCH)], osem)
    out_desc(nch - 2).wait()
    out_desc(nch - 1).wait()


def _combine(slots, og):
    mesh = plsc.VectorSubcoreMesh(core_axis_name="c", subcore_axis_name="s",
                                  num_cores=2, num_subcores=16)
    f = pl.kernel(
        _combine_body,
        out_type=jax.ShapeDtypeStruct((N_, D_), jnp.float32),
        mesh=mesh,
        compiler_params=pltpu.CompilerParams(needs_layout_passes=False),
        scratch_types=[
            pltpu.VMEM((2, 2 * _TCH), jnp.int32),
            pltpu.VMEM((2, 2 * _TCH), jnp.int32),
            pltpu.VMEM((2, 2 * _TCH, D_), jnp.float32),
            pltpu.VMEM((2, _TCH, D_), jnp.float32),
            pltpu.SemaphoreType.DMA,
            pltpu.SemaphoreType.DMA,
            pltpu.SemaphoreType.DMA,
        ],
    )
    return f(slots, og)


def kernel(x, Wr, W1, b1, W2, b2):
    xf = x.reshape(N_, D_)
    idx, w, lb = _router(xf, Wr)
    xg, ws, slots = _dispatch(idx.reshape(NK), w.reshape(NK), xf)
    og = _ffn(xg, W1, b1, W2, b2, ws.reshape(NK, 1))
    out = _combine(slots, og)
    return out.reshape(B_, S_, D_), lb[0, 0]


# speedup vs baseline: 1.1695x; 1.1695x over previous
import functools

import jax
import jax.numpy as jnp
from jax import lax
from jax.experimental import pallas as pl
from jax.experimental.pallas import tpu as pltpu
from jax.experimental.pallas import tpu_sc as plsc

B_, S_, D_, F_, E_, K_ = 2, 2048, 1024, 4096, 8, 2
N_ = B_ * S_
NK = N_ * K_
CAP = int(2.0 * N_ / E_)
FB = 1024
NFB = F_ // FB
DUMP = NK


_TB = 1024


def _router_body(x_ref, wr_ref, idx_ref, w_ref, lb_ref, cnt_acc):
    i = pl.program_id(0)
    x = x_ref[...]
    wr = wr_ref[...]
    logits = lax.dot_general(x, wr, (((1,), (1,)), ((), ())),
                             preferred_element_type=jnp.float32)
    iota = lax.broadcasted_iota(jnp.int32, logits.shape, 1)
    l1 = jnp.max(logits, axis=1, keepdims=True)
    am1 = jnp.min(jnp.where(logits == l1, iota, E_), axis=1, keepdims=True)
    masked = jnp.where(iota == am1, -jnp.inf, logits)
    l2 = jnp.max(masked, axis=1, keepdims=True)
    am2 = jnp.min(jnp.where(masked == l2, iota, E_), axis=1, keepdims=True)
    z = jnp.exp(l2 - l1)
    s = 1.0 / (1.0 + z)
    idx_ref[...] = jnp.concatenate([am1, am2], axis=1)
    w_ref[...] = jnp.concatenate([s, z * s], axis=1)
    oh = (iota == am1).astype(jnp.float32) + (iota == am2).astype(jnp.float32)
    c = jnp.sum(oh, axis=0, keepdims=True)

    @pl.when(i == 0)
    def _():
        cnt_acc[...] = c

    @pl.when(i > 0)
    def _():
        cnt_acc[...] += c

    @pl.when(i == pl.num_programs(0) - 1)
    def _():
        cc = cnt_acc[...]
        ideal = jnp.float32(N_ * K_ / E_)
        lb_ref[...] = (jnp.sum((cc - ideal) ** 2) / jnp.float32(N_) ** 2
                       ).reshape(1, 1)


def _router(xf, Wr):
    return pl.pallas_call(
        _router_body,
        grid=(N_ // _TB,),
        in_specs=[pl.BlockSpec((_TB, D_), lambda i: (i, 0)),
                  pl.BlockSpec((E_, D_), lambda i: (0, 0))],
        out_specs=[pl.BlockSpec((_TB, K_), lambda i: (i, 0)),
                   pl.BlockSpec((_TB, K_), lambda i: (i, 0)),
                   pl.BlockSpec((1, 1), lambda i: (0, 0))],
        out_shape=[jax.ShapeDtypeStruct((N_, K_), jnp.int32),
                   jax.ShapeDtypeStruct((N_, K_), jnp.float32),
                   jax.ShapeDtypeStruct((1, 1), jnp.float32)],
        scratch_shapes=[pltpu.VMEM((1, E_), jnp.float32)],
    )(xf, Wr)


_CH = NK // 16
_ZSPAN = 528
_SH = 16 * _ZSPAN


_GR = 32
_GPT = NK // 32


def _dispatch_body(idx_hbm, w_hbm, xf_hbm, xg_hbm, w_out, slot_out,
                   ich, wch, dvals, rvals, svals, cvm, cnt_all, zvi, zvf,
                   gidx, gbuf, gs0, gs1, osem,
                   sh_cnt, sh_rows, sh_w):
    cid = lax.axis_index("c")
    wid = lax.axis_index("s")
    base_slot = wid * _CH

    pltpu.sync_copy(idx_hbm.at[pl.ds(base_slot, _CH)], ich)
    pltpu.sync_copy(w_hbm.at[pl.ds(base_slot, _CH)], wch)

    lanes = lax.iota(jnp.int32, 16)

    def hist_step(i, cnt):
        v = ich[pl.ds(i * 16, 16)]
        for e in range(E_):
            pc = jnp.sum((v == e).astype(jnp.int32))
            cnt = cnt + jnp.where(lanes == e, pc, 0)
        return cnt

    cnt = lax.fori_loop(0, _CH // 16, hist_step, jnp.zeros((16,), jnp.int32))
    cvm[...] = cnt
    pltpu.sync_copy(cvm, sh_cnt.at[pl.ds(wid * 16, 16)])

    def zfill(k, _):
        zvi[pl.ds(k * 16, 16)] = jnp.zeros((16,), jnp.int32)
        zvf[pl.ds(k * 16, 16)] = jnp.zeros((16,), jnp.float32)
        return 0

    lax.fori_loop(0, _ZSPAN // 16, zfill, 0)
    pltpu.sync_copy(zvi, sh_rows.at[pl.ds(wid * _ZSPAN, _ZSPAN)])
    pltpu.sync_copy(zvf, sh_w.at[pl.ds(wid * _ZSPAN, _ZSPAN)])

    plsc.subcore_barrier()

    pltpu.sync_copy(sh_cnt, cnt_all)

    base_v = jnp.zeros((16,), jnp.int32)
    for t in range(16):
        base_v = base_v + jnp.where(t < wid, cnt_all[pl.ds(t * 16, 16)], 0)
    offs = tuple(jnp.sum(jnp.where(lanes == e, base_v, 0)) for e in range(E_))

    def pc_step(i, offs):
        v = ich[pl.ds(i * 16, 16)]
        rank = jnp.zeros((16,), jnp.int32)
        new = []
        for e in range(E_):
            m = v == e
            mi = m.astype(jnp.int32)
            cs = plsc.cumsum(mi)
            rank = rank + jnp.where(m, cs - 1 + offs[e], 0)
            new.append(offs[e] + jnp.sum(mi))
        keep = rank < CAP
        dest = jnp.where(keep, v * CAP + rank, DUMP)
        dvals[i // 8, pl.ds((i % 8) * 16, 16)] = dest
        svals[pl.ds(i * 16, 16)] = dest
        slotid = base_slot + i * 16 + lanes
        rvals[pl.ds(i * 16, 16)] = lax.shift_right_logical(slotid, 1)
        return tuple(new)

    lax.fori_loop(0, _CH // 16, pc_step, offs)

    @pl.when(cid == 0)
    def _():
        pltpu.sync_copy(svals, slot_out.at[pl.ds(base_slot, _CH)])

    plsc.subcore_barrier()
    for j in range(_CH // 128):
        pltpu.sync_copy(rvals.at[pl.ds(j * 128, 128)],
                        sh_rows.at[dvals.at[j]], add=True)
        pltpu.sync_copy(wch.at[pl.ds(j * 128, 128)],
                        sh_w.at[dvals.at[j]], add=True)
    plsc.subcore_barrier()

    @pl.when(cid == 0)
    def _():
        pltpu.sync_copy(sh_w.at[pl.ds(base_slot, _CH)],
                        w_out.at[pl.ds(base_slot, _CH)])

    gwid = wid * 2 + cid
    gbase = gwid * _GPT
    nch = _GPT // _GR
    gsem = (gs0, gs1)

    def start_gather(j):
        k = j % 2
        pltpu.sync_copy(sh_rows.at[pl.ds(gbase + j * _GR, _GR)], gidx.at[k])
        pltpu.async_copy(xf_hbm.at[gidx.at[k]], gbuf.at[k], gsem[k])

    def out_desc(j):
        k = j % 2
        return pltpu.make_async_copy(
            gbuf.at[k], xg_hbm.at[pl.ds(gbase + j * _GR, _GR)], osem)

    start_gather(0)
    for j in range(nch):
        k = j % 2
        if j + 1 < nch:
            if j >= 1:
                out_desc(j - 1).wait()
            start_gather(j + 1)
        pltpu.make_async_copy(xf_hbm.at[gidx.at[k]], gbuf.at[k],
                              gsem[k]).wait()
        pltpu.async_copy(gbuf.at[k], xg_hbm.at[pl.ds(gbase + j * _GR, _GR)],
                         osem)
    out_desc(nch - 2).wait()
    out_desc(nch - 1).wait()


def _dispatch(idx_flat, w_flat, xf):
    mesh = plsc.VectorSubcoreMesh(core_axis_name="c", subcore_axis_name="s",
                                  num_cores=2, num_subcores=16)
    f = pl.kernel(
        _dispatch_body,
        out_type=[jax.ShapeDtypeStruct((NK, D_), jnp.float32),
                  jax.ShapeDtypeStruct((NK,), jnp.float32),
                  jax.ShapeDtypeStruct((NK,), jnp.int32)],
        mesh=mesh,
        compiler_params=pltpu.CompilerParams(needs_layout_passes=False),
        scratch_types=[
            pltpu.VMEM((_CH,), jnp.int32),
            pltpu.VMEM((_CH,), jnp.float32),
            pltpu.VMEM((_CH // 128, 128), jnp.int32),
            pltpu.VMEM((_CH,), jnp.int32),
            pltpu.VMEM((_CH,), jnp.int32),
            pltpu.VMEM((16,), jnp.int32),
            pltpu.VMEM((256,), jnp.int32),
            pltpu.VMEM((_ZSPAN,), jnp.int32),
            pltpu.VMEM((_ZSPAN,), jnp.float32),
            pltpu.VMEM((2, _GR), jnp.int32),
            pltpu.VMEM((2, _GR, D_), jnp.float32),
            pltpu.SemaphoreType.DMA,
            pltpu.SemaphoreType.DMA,
            pltpu.SemaphoreType.DMA,
            pltpu.VMEM_SHARED((256,), jnp.int32),
            pltpu.VMEM_SHARED((_SH,), jnp.int32),
            pltpu.VMEM_SHARED((_SH,), jnp.float32),
        ],
    )
    return f(idx_flat, w_flat, xf)


_SQRT_HALF = 0.7071067811865476


_FB2 = 2048
_NFB2 = F_ // _FB2
_NSUB = 2
_SB = _FB2 // _NSUB


def _ffn_body(xg_ref, w1_ref, b1_ref, w2_ref, b2_ref, ws_ref, og_ref, acc):
    fb = pl.program_id(1)
    x = xg_ref[...].astype(jnp.bfloat16)
    part = None
    for sub in range(_NSUB):
        w1 = w1_ref[0, pl.ds(sub * _SB, _SB), :].astype(jnp.bfloat16)
        h = lax.dot_general(x, w1, (((1,), (1,)), ((), ())),
                            preferred_element_type=jnp.float32)
        h = h + b1_ref[0, :, pl.ds(sub * _SB, _SB)]
        h = 0.5 * h * (1.0 + lax.erf(h * _SQRT_HALF))
        w2 = w2_ref[0, :, pl.ds(sub * _SB, _SB)].astype(jnp.bfloat16)
        p = lax.dot_general(h.astype(jnp.bfloat16), w2,
                            (((1,), (1,)), ((), ())),
                            preferred_element_type=jnp.float32)
        part = p if part is None else part + p

    @pl.when(fb == 0)
    def _():
        acc[...] = part

    @pl.when(fb == _NFB2 - 1)
    def _():
        og_ref[...] = (acc[...] + part + b2_ref[0]) * ws_ref[...]


def _ffn(xg, W1, b1, W2, b2, ws):
    return pl.pallas_call(
        _ffn_body,
        grid=(E_, _NFB2),
        in_specs=[
            pl.BlockSpec((CAP, D_), lambda e, fb: (e, 0)),
            pl.BlockSpec((1, _FB2, D_), lambda e, fb: (e, fb, 0)),
            pl.BlockSpec((1, 1, _FB2), lambda e, fb: (e * _NFB2 + fb, 0, 0)),
            pl.BlockSpec((1, D_, _FB2), lambda e, fb: (e, 0, fb)),
            pl.BlockSpec((1, 1, D_), lambda e, fb: (e, 0, 0)),
            pl.BlockSpec((CAP, 1), lambda e, fb: (e, 0)),
        ],
        out_specs=pl.BlockSpec((CAP, D_), lambda e, fb: (e, 0)),
        out_shape=jax.ShapeDtypeStruct((NK, D_), jnp.float32),
        scratch_shapes=[pltpu.VMEM((CAP, D_), jnp.float32)],
        compiler_params=pltpu.CompilerParams(
            dimension_semantics=("arbitrary", "arbitrary"),
            vmem_limit_bytes=63 * 1024 * 1024),
    )(xg, W1, b1.reshape(E_ * _NFB2, 1, _FB2), W2, b2.reshape(E_, 1, D_), ws)


_TPT = N_ // 32
_TCH = 16


def _combine_body(slots_hbm, og_hbm, out_hbm, sbufr, idxv, buf, obuf,
                  gs0, gs1, osem):
    cid = lax.axis_index("c")
    sid = lax.axis_index("s")
    wid = sid * 2 + cid
    tbase = wid * _TPT
    lanes = lax.iota(jnp.int32, 16)
    nch = _TPT // _TCH
    gsem = (gs0, gs1)

    def start_gather(j):
        k = j % 2
        sb = (tbase + j * _TCH) * 2
        pltpu.sync_copy(slots_hbm.at[pl.ds(sb, 2 * _TCH)], sbufr.at[k])
        sv0 = sbufr[k, pl.ds(0, 16)]
        sv1 = sbufr[k, pl.ds(16, 16)]
        idxv[k, pl.ds(0, 16)] = jnp.minimum(sv0, NK - 1)
        idxv[k, pl.ds(16, 16)] = jnp.minimum(sv1, NK - 1)
        pltpu.async_copy(og_hbm.at[idxv.at[k]], buf.at[k], gsem[k])

    def out_desc(j):
        k = j % 2
        return pltpu.make_async_copy(
            obuf.at[k], out_hbm.at[pl.ds(tbase + j * _TCH, _TCH)], osem)

    start_gather(0)
    for j in range(nch):
        k = j % 2
        if j + 1 < nch:
            start_gather(j + 1)
        pltpu.make_async_copy(og_hbm.at[idxv.at[k]], buf.at[k], gsem[k]).wait()
        if j >= 2:
            out_desc(j - 2).wait()
        sv0 = sbufr[k, pl.ds(0, 16)]
        sv1 = sbufr[k, pl.ds(16, 16)]

        def token(t, carry):
            p0 = 2 * t
            p1 = 2 * t + 1
            s0 = jnp.where(
                t < 8,
                jnp.max(jnp.where(lanes == p0, sv0, -1)),
                jnp.max(jnp.where(lanes == p0 - 16, sv1, -1)))
            s1 = jnp.where(
                t < 8,
                jnp.max(jnp.where(lanes == p1, sv0, -1)),
                jnp.max(jnp.where(lanes == p1 - 16, sv1, -1)))
            b0 = s0 < NK
            b1 = s1 < NK
            for c in range(D_ // 16):
                r0 = buf[k, p0, pl.ds(c * 16, 16)]
                r1 = buf[k, p1, pl.ds(c * 16, 16)]
                obuf[k, t, pl.ds(c * 16, 16)] = (
                    jnp.where(b0, r0, 0.0) + jnp.where(b1, r1, 0.0))
            return carry

        lax.fori_loop(0, _TCH, token, 0)
        pltpu.async_copy(obuf.at[k],
                         out_hbm.at[pl.ds(tbase + j * _TCH, _TCH)], osem)
    out_desc(nch - 2).wait()
    out_desc(nch - 1).wait()


def _combine(slots, og):
    mesh = plsc.VectorSubcoreMesh(core_axis_name="c", subcore_axis_name="s",
                                  num_cores=2, num_subcores=16)
    f = pl.kernel(
        _combine_body,
        out_type=jax.ShapeDtypeStruct((N_, D_), jnp.float32),
        mesh=mesh,
        compiler_params=pltpu.CompilerParams(needs_layout_passes=False),
        scratch_types=[
            pltpu.VMEM((2, 2 * _TCH), jnp.int32),
            pltpu.VMEM((2, 2 * _TCH), jnp.int32),
            pltpu.VMEM((2, 2 * _TCH, D_), jnp.float32),
            pltpu.VMEM((2, _TCH, D_), jnp.float32),
            pltpu.SemaphoreType.DMA,
            pltpu.SemaphoreType.DMA,
            pltpu.SemaphoreType.DMA,
        ],
    )
    return f(slots, og)


def kernel(x, Wr, W1, b1, W2, b2):
    xf = x.reshape(N_, D_)
    idx, w, lb = _router(xf, Wr)
    xg, ws, slots = _dispatch(idx.reshape(NK), w.reshape(NK), xf)
    og = _ffn(xg, W1, b1, W2, b2, ws.reshape(NK, 1))
    out = _combine(slots, og)
    return out.reshape(B_, S_, D_), lb[0, 0]


# speedup vs baseline: 1.1696x; 1.0001x over previous
import functools

import jax
import jax.numpy as jnp
from jax import lax
from jax.experimental import pallas as pl
from jax.experimental.pallas import tpu as pltpu
from jax.experimental.pallas import tpu_sc as plsc

B_, S_, D_, F_, E_, K_ = 2, 2048, 1024, 4096, 8, 2
N_ = B_ * S_
NK = N_ * K_
CAP = int(2.0 * N_ / E_)
FB = 1024
NFB = F_ // FB
DUMP = NK


_TB = 1024


def _router_body(x_ref, wr_ref, idx_ref, w_ref, lb_ref, cnt_acc):
    i = pl.program_id(0)
    x = x_ref[...]
    wr = wr_ref[...]
    logits = lax.dot_general(x, wr, (((1,), (1,)), ((), ())),
                             preferred_element_type=jnp.float32)
    iota = lax.broadcasted_iota(jnp.int32, logits.shape, 1)
    l1 = jnp.max(logits, axis=1, keepdims=True)
    am1 = jnp.min(jnp.where(logits == l1, iota, E_), axis=1, keepdims=True)
    masked = jnp.where(iota == am1, -jnp.inf, logits)
    l2 = jnp.max(masked, axis=1, keepdims=True)
    am2 = jnp.min(jnp.where(masked == l2, iota, E_), axis=1, keepdims=True)
    z = jnp.exp(l2 - l1)
    s = 1.0 / (1.0 + z)
    idx_ref[...] = jnp.concatenate([am1, am2], axis=1)
    w_ref[...] = jnp.concatenate([s, z * s], axis=1)
    oh = (iota == am1).astype(jnp.float32) + (iota == am2).astype(jnp.float32)
    c = jnp.sum(oh, axis=0, keepdims=True)

    @pl.when(i == 0)
    def _():
        cnt_acc[...] = c

    @pl.when(i > 0)
    def _():
        cnt_acc[...] += c

    @pl.when(i == pl.num_programs(0) - 1)
    def _():
        cc = cnt_acc[...]
        ideal = jnp.float32(N_ * K_ / E_)
        lb_ref[...] = (jnp.sum((cc - ideal) ** 2) / jnp.float32(N_) ** 2
                       ).reshape(1, 1)


def _router(xf, Wr):
    return pl.pallas_call(
        _router_body,
        grid=(N_ // _TB,),
        in_specs=[pl.BlockSpec((_TB, D_), lambda i: (i, 0)),
                  pl.BlockSpec((E_, D_), lambda i: (0, 0))],
        out_specs=[pl.BlockSpec((_TB, K_), lambda i: (i, 0)),
                   pl.BlockSpec((_TB, K_), lambda i: (i, 0)),
                   pl.BlockSpec((1, 1), lambda i: (0, 0))],
        out_shape=[jax.ShapeDtypeStruct((N_, K_), jnp.int32),
                   jax.ShapeDtypeStruct((N_, K_), jnp.float32),
                   jax.ShapeDtypeStruct((1, 1), jnp.float32)],
        scratch_shapes=[pltpu.VMEM((1, E_), jnp.float32)],
    )(xf, Wr)


_CH = NK // 16
_ZSPAN = 528
_SH = 16 * _ZSPAN


_GR = 32
_GPT = NK // 32


def _dispatch_body(idx_hbm, w_hbm, xf_hbm, xg_hbm, w_out, slot_out,
                   ich, wch, dvals, rvals, svals, cvm, cnt_all, zvi, zvf,
                   gidx, gbuf, gs0, gs1, osem,
                   sh_cnt, sh_rows, sh_w):
    cid = lax.axis_index("c")
    wid = lax.axis_index("s")
    base_slot = wid * _CH

    pltpu.sync_copy(idx_hbm.at[pl.ds(base_slot, _CH)], ich)
    pltpu.sync_copy(w_hbm.at[pl.ds(base_slot, _CH)], wch)

    lanes = lax.iota(jnp.int32, 16)

    def hist_step(i, cnt):
        v = ich[pl.ds(i * 16, 16)]
        for e in range(E_):
            pc = jnp.sum((v == e).astype(jnp.int32))
            cnt = cnt + jnp.where(lanes == e, pc, 0)
        return cnt

    cnt = lax.fori_loop(0, _CH // 16, hist_step, jnp.zeros((16,), jnp.int32))
    cvm[...] = cnt
    pltpu.sync_copy(cvm, sh_cnt.at[pl.ds(wid * 16, 16)])

    def zfill(k, _):
        zvi[pl.ds(k * 16, 16)] = jnp.zeros((16,), jnp.int32)
        zvf[pl.ds(k * 16, 16)] = jnp.zeros((16,), jnp.float32)
        return 0

    lax.fori_loop(0, _ZSPAN // 16, zfill, 0)
    pltpu.sync_copy(zvi, sh_rows.at[pl.ds(wid * _ZSPAN, _ZSPAN)])
    pltpu.sync_copy(zvf, sh_w.at[pl.ds(wid * _ZSPAN, _ZSPAN)])

    plsc.subcore_barrier()

    pltpu.sync_copy(sh_cnt, cnt_all)

    base_v = jnp.zeros((16,), jnp.int32)
    for t in range(16):
        base_v = base_v + jnp.where(t < wid, cnt_all[pl.ds(t * 16, 16)], 0)
    offs = tuple(jnp.sum(jnp.where(lanes == e, base_v, 0)) for e in range(E_))

    def pc_step(i, offs):
        v = ich[pl.ds(i * 16, 16)]
        rank = jnp.zeros((16,), jnp.int32)
        new = []
        for e in range(E_):
            m = v == e
            mi = m.astype(jnp.int32)
            cs = plsc.cumsum(mi)
            rank = rank + jnp.where(m, cs - 1 + offs[e], 0)
            new.append(offs[e] + jnp.sum(mi))
        keep = rank < CAP
        dest = jnp.where(keep, v * CAP + rank, DUMP)
        dvals[i // 8, pl.ds((i % 8) * 16, 16)] = dest
        svals[pl.ds(i * 16, 16)] = dest
        slotid = base_slot + i * 16 + lanes
        rvals[pl.ds(i * 16, 16)] = lax.shift_right_logical(slotid, 1)
        return tuple(new)

    lax.fori_loop(0, _CH // 16, pc_step, offs)

    @pl.when(cid == 0)
    def _():
        pltpu.sync_copy(svals, slot_out.at[pl.ds(base_slot, _CH)])

    plsc.subcore_barrier()
    for j in range(_CH // 128):
        pltpu.sync_copy(rvals.at[pl.ds(j * 128, 128)],
                        sh_rows.at[dvals.at[j]], add=True)
        pltpu.sync_copy(wch.at[pl.ds(j * 128, 128)],
                        sh_w.at[dvals.at[j]], add=True)
    plsc.subcore_barrier()

    @pl.when(cid == 0)
    def _():
        pltpu.sync_copy(sh_w.at[pl.ds(base_slot, _CH)],
                        w_out.at[pl.ds(base_slot, _CH)])

    gwid = wid * 2 + cid
    gbase = gwid * _GPT
    nch = _GPT // _GR
    gsem = (gs0, gs1)

    def start_gather(j):
        k = j % 2
        pltpu.sync_copy(sh_rows.at[pl.ds(gbase + j * _GR, _GR)], gidx.at[k])
        pltpu.async_copy(xf_hbm.at[gidx.at[k]], gbuf.at[k], gsem[k])

    def out_desc(j):
        k = j % 2
        return pltpu.make_async_copy(
            gbuf.at[k], xg_hbm.at[pl.ds(gbase + j * _GR, _GR)], osem)

    start_gather(0)
    for j in range(nch):
        k = j % 2
        if j + 1 < nch:
            if j >= 1:
                out_desc(j - 1).wait()
            start_gather(j + 1)
        pltpu.make_async_copy(xf_hbm.at[gidx.at[k]], gbuf.at[k],
                              gsem[k]).wait()
        pltpu.async_copy(gbuf.at[k], xg_hbm.at[pl.ds(gbase + j * _GR, _GR)],
                         osem)
    out_desc(nch - 2).wait()
    out_desc(nch - 1).wait()


def _dispatch(idx_flat, w_flat, xf):
    mesh = plsc.VectorSubcoreMesh(core_axis_name="c", subcore_axis_name="s",
                                  num_cores=2, num_subcores=16)
    f = pl.kernel(
        _dispatch_body,
        out_type=[jax.ShapeDtypeStruct((NK, D_), jnp.float32),
                  jax.ShapeDtypeStruct((NK,), jnp.float32),
                  jax.ShapeDtypeStruct((NK,), jnp.int32)],
        mesh=mesh,
        compiler_params=pltpu.CompilerParams(needs_layout_passes=False),
        scratch_types=[
            pltpu.VMEM((_CH,), jnp.int32),
            pltpu.VMEM((_CH,), jnp.float32),
            pltpu.VMEM((_CH // 128, 128), jnp.int32),
            pltpu.VMEM((_CH,), jnp.int32),
            pltpu.VMEM((_CH,), jnp.int32),
            pltpu.VMEM((16,), jnp.int32),
            pltpu.VMEM((256,), jnp.int32),
            pltpu.VMEM((_ZSPAN,), jnp.int32),
            pltpu.VMEM((_ZSPAN,), jnp.float32),
            pltpu.VMEM((2, _GR), jnp.int32),
            pltpu.VMEM((2, _GR, D_), jnp.float32),
            pltpu.SemaphoreType.DMA,
            pltpu.SemaphoreType.DMA,
            pltpu.SemaphoreType.DMA,
            pltpu.VMEM_SHARED((256,), jnp.int32),
            pltpu.VMEM_SHARED((_SH,), jnp.int32),
            pltpu.VMEM_SHARED((_SH,), jnp.float32),
        ],
    )
    return f(idx_flat, w_flat, xf)


_SQRT_HALF = 0.7071067811865476


_FB2 = 2048
_NFB2 = F_ // _FB2
_NSUB = 1
_SB = _FB2 // _NSUB


def _ffn_body(xg_ref, w1_ref, b1_ref, w2_ref, b2_ref, ws_ref, og_ref, acc):
    fb = pl.program_id(1)
    x = xg_ref[...].astype(jnp.bfloat16)
    part = None
    for sub in range(_NSUB):
        w1 = w1_ref[0, pl.ds(sub * _SB, _SB), :].astype(jnp.bfloat16)
        h = lax.dot_general(x, w1, (((1,), (1,)), ((), ())),
                            preferred_element_type=jnp.float32)
        h = h + b1_ref[0, :, pl.ds(sub * _SB, _SB)]
        h = 0.5 * h * (1.0 + lax.erf(h * _SQRT_HALF))
        w2 = w2_ref[0, :, pl.ds(sub * _SB, _SB)].astype(jnp.bfloat16)
        p = lax.dot_general(h.astype(jnp.bfloat16), w2,
                            (((1,), (1,)), ((), ())),
                            preferred_element_type=jnp.float32)
        part = p if part is None else part + p

    @pl.when(fb == 0)
    def _():
        acc[...] = part

    @pl.when(fb == _NFB2 - 1)
    def _():
        og_ref[...] = (acc[...] + part + b2_ref[0]) * ws_ref[...]


def _ffn(xg, W1, b1, W2, b2, ws):
    return pl.pallas_call(
        _ffn_body,
        grid=(E_, _NFB2),
        in_specs=[
            pl.BlockSpec((CAP, D_), lambda e, fb: (e, 0)),
            pl.BlockSpec((1, _FB2, D_), lambda e, fb: (e, fb, 0)),
            pl.BlockSpec((1, 1, _FB2), lambda e, fb: (e * _NFB2 + fb, 0, 0)),
            pl.BlockSpec((1, D_, _FB2), lambda e, fb: (e, 0, fb)),
            pl.BlockSpec((1, 1, D_), lambda e, fb: (e, 0, 0)),
            pl.BlockSpec((CAP, 1), lambda e, fb: (e, 0)),
        ],
        out_specs=pl.BlockSpec((CAP, D_), lambda e, fb: (e, 0)),
        out_shape=jax.ShapeDtypeStruct((NK, D_), jnp.float32),
        scratch_shapes=[pltpu.VMEM((CAP, D_), jnp.float32)],
        compiler_params=pltpu.CompilerParams(
            dimension_semantics=("arbitrary", "arbitrary"),
            vmem_limit_bytes=63 * 1024 * 1024),
    )(xg, W1, b1.reshape(E_ * _NFB2, 1, _FB2), W2, b2.reshape(E_, 1, D_), ws)


_TPT = N_ // 32
_TCH = 16


def _combine_body(slots_hbm, og_hbm, out_hbm, sbufr, idxv, buf, obuf,
                  gs0, gs1, osem):
    cid = lax.axis_index("c")
    sid = lax.axis_index("s")
    wid = sid * 2 + cid
    tbase = wid * _TPT
    lanes = lax.iota(jnp.int32, 16)
    nch = _TPT // _TCH
    gsem = (gs0, gs1)

    def start_gather(j):
        k = j % 2
        sb = (tbase + j * _TCH) * 2
        pltpu.sync_copy(slots_hbm.at[pl.ds(sb, 2 * _TCH)], sbufr.at[k])
        sv0 = sbufr[k, pl.ds(0, 16)]
        sv1 = sbufr[k, pl.ds(16, 16)]
        idxv[k, pl.ds(0, 16)] = jnp.minimum(sv0, NK - 1)
        idxv[k, pl.ds(16, 16)] = jnp.minimum(sv1, NK - 1)
        pltpu.async_copy(og_hbm.at[idxv.at[k]], buf.at[k], gsem[k])

    def out_desc(j):
        k = j % 2
        return pltpu.make_async_copy(
            obuf.at[k], out_hbm.at[pl.ds(tbase + j * _TCH, _TCH)], osem)

    start_gather(0)
    for j in range(nch):
        k = j % 2
        if j + 1 < nch:
            start_gather(j + 1)
        pltpu.make_async_copy(og_hbm.at[idxv.at[k]], buf.at[k], gsem[k]).wait()
        if j >= 2:
            out_desc(j - 2).wait()
        sv0 = sbufr[k, pl.ds(0, 16)]
        sv1 = sbufr[k, pl.ds(16, 16)]

        def token(t, carry):
            p0 = 2 * t
            p1 = 2 * t + 1
            s0 = jnp.where(
                t < 8,
                jnp.max(jnp.where(lanes == p0, sv0, -1)),
                jnp.max(jnp.where(lanes == p0 - 16, sv1, -1)))
            s1 = jnp.where(
                t < 8,
                jnp.max(jnp.where(lanes == p1, sv0, -1)),
                jnp.max(jnp.where(lanes == p1 - 16, sv1, -1)))
            b0 = s0 < NK
            b1 = s1 < NK
            for c in range(D_ // 16):
                r0 = buf[k, p0, pl.ds(c * 16, 16)]
                r1 = buf[k, p1, pl.ds(c * 16, 16)]
                obuf[k, t, pl.ds(c * 16, 16)] = (
                    jnp.where(b0, r0, 0.0) + jnp.where(b1, r1, 0.0))
            return carry

        lax.fori_loop(0, _TCH, token, 0)
        pltpu.async_copy(obuf.at[k],
                         out_hbm.at[pl.ds(tbase + j * _TCH, _TCH)], osem)
    out_desc(nch - 2).wait()
    out_desc(nch - 1).wait()


def _combine(slots, og):
    mesh = plsc.VectorSubcoreMesh(core_axis_name="c", subcore_axis_name="s",
                                  num_cores=2, num_subcores=16)
    f = pl.kernel(
        _combine_body,
        out_type=jax.ShapeDtypeStruct((N_, D_), jnp.float32),
        mesh=mesh,
        compiler_params=pltpu.CompilerParams(needs_layout_passes=False),
        scratch_types=[
            pltpu.VMEM((2, 2 * _TCH), jnp.int32),
            pltpu.VMEM((2, 2 * _TCH), jnp.int32),
            pltpu.VMEM((2, 2 * _TCH, D_), jnp.float32),
            pltpu.VMEM((2, _TCH, D_), jnp.float32),
            pltpu.SemaphoreType.DMA,
            pltpu.SemaphoreType.DMA,
            pltpu.SemaphoreType.DMA,
        ],
    )
    return f(slots, og)


def kernel(x, Wr, W1, b1, W2, b2):
    xf = x.reshape(N_, D_)
    idx, w, lb = _router(xf, Wr)
    xg, ws, slots = _dispatch(idx.reshape(NK), w.reshape(NK), xf)
    og = _ffn(xg, W1, b1, W2, b2, ws.reshape(NK, 1))
    out = _combine(slots, og)
    return out.reshape(B_, S_, D_), lb[0, 0]
